# baseline (speedup 1.0000x reference)
import functools

import jax
import jax.numpy as jnp
import numpy as np
from jax import lax
from jax.experimental import pallas as pl
from jax.experimental.pallas import tpu as pltpu
from jax.experimental.pallas import tpu_sc as plsc

N_NODES = 10000
NPAD = 10240
D_IN = 128
D_HID = 64
D_OUT = 16
N_CLASSES = 10
N_EDGES = 320000

NC, NS = 2, 16
NW = NC * NS
CHUNK = 80
K_CHUNKS = 125
NBUF = 10

ROWS_PER_TILE = NPAD // NS
COPIES_PER_TILE = ROWS_PER_TILE // CHUNK

_MESH = plsc.VectorSubcoreMesh(core_axis_name="c", subcore_axis_name="s")
_SC_PARAMS = pltpu.CompilerParams(use_tc_tiling_on_sc=False)
_SC_PARAMS_NL = pltpu.CompilerParams(use_tc_tiling_on_sc=False,
                                     needs_layout_passes=False)


def _make_agg(width):
    @functools.partial(
        pl.kernel,
        out_type=(jax.ShapeDtypeStruct((NPAD, width), jnp.float32),
                  jax.ShapeDtypeStruct((NPAD, width), jnp.float32)),
        mesh=_MESH,
        scratch_types=[
            pltpu.VMEM((K_CHUNKS, CHUNK), jnp.int32),
            pltpu.VMEM((K_CHUNKS, CHUNK), jnp.int32),
            [pltpu.VMEM((CHUNK, width), jnp.float32) for _ in range(NBUF)],
            pltpu.VMEM_SHARED((NPAD, width), jnp.float32),
            [pltpu.SemaphoreType.DMA for _ in range(NBUF)],
            [pltpu.SemaphoreType.DMA for _ in range(NBUF)],
        ],
        compiler_params=_SC_PARAMS,
    )
    def agg(g_hbm, src_hbm, dst_hbm, zeros_hbm, outa_hbm, outb_hbm,
            src_v, dst_v, bufs, acc_sh, gsems, ssems):
        cid = lax.axis_index("c")
        sid = lax.axis_index("s")
        wid = sid * NC + cid

        pltpu.sync_copy(zeros_hbm, bufs[0])
        for b in range(COPIES_PER_TILE):
            r0 = sid * ROWS_PER_TILE + b * CHUNK
            pltpu.sync_copy(bufs[0], acc_sh.at[pl.ds(r0, CHUNK), :])

        pltpu.sync_copy(src_hbm.at[wid], src_v)
        pltpu.sync_copy(dst_hbm.at[wid], dst_v)

        for b in range(NBUF):
            pltpu.async_copy(g_hbm.at[src_v.at[b]], bufs[b], gsems[b])
        plsc.subcore_barrier()

        def rnd(i, carry):
            for b in range(NBUF):
                c = i * NBUF + b
                pltpu.make_async_copy(
                    g_hbm.at[src_v.at[c]], bufs[b], gsems[b]).wait()
                pltpu.async_copy(
                    bufs[b], acc_sh.at[dst_v.at[c]], ssems[b], add=True)
            for b in range(NBUF):
                c2 = (i + 1) * NBUF + b
                pltpu.make_async_copy(
                    bufs[b], acc_sh.at[dst_v.at[0]], ssems[b]).wait()

                @pl.when(c2 < K_CHUNKS)
                def _():
                    pltpu.async_copy(g_hbm.at[src_v.at[c2]], bufs[b], gsems[b])
            return carry

        lax.fori_loop(0, K_CHUNKS // NBUF, rnd, 0)
        for b in range(K_CHUNKS % NBUF):
            c = (K_CHUNKS // NBUF) * NBUF + b
            pltpu.make_async_copy(
                g_hbm.at[src_v.at[c]], bufs[b], gsems[b]).wait()
            pltpu.async_copy(
                bufs[b], acc_sh.at[dst_v.at[c]], ssems[b], add=True)
        for b in range(K_CHUNKS % NBUF):
            pltpu.make_async_copy(
                bufs[b], acc_sh.at[dst_v.at[0]], ssems[b]).wait()
        plsc.subcore_barrier()

        for b in range(COPIES_PER_TILE):
            r0 = sid * ROWS_PER_TILE + b * CHUNK
            pltpu.sync_copy(acc_sh.at[pl.ds(r0, CHUNK), :], bufs[0])

            @pl.when(cid == 0)
            def _():
                pltpu.sync_copy(bufs[0], outa_hbm.at[pl.ds(r0, CHUNK), :])

            @pl.when(cid == 1)
            def _():
                pltpu.sync_copy(bufs[0], outb_hbm.at[pl.ds(r0, CHUNK), :])

    return agg


_agg64 = _make_agg(D_HID)
_agg16 = _make_agg(D_OUT)

_GROUPS_PER_TILE = ROWS_PER_TILE // 16


@functools.partial(
    pl.kernel,
    out_type=jax.ShapeDtypeStruct((NC, NPAD), jnp.float32),
    mesh=_MESH,
    scratch_types=[
        pltpu.VMEM((K_CHUNKS, CHUNK), jnp.int32),
        pltpu.VMEM((CHUNK, D_OUT), jnp.float32),
        pltpu.VMEM((ROWS_PER_TILE, D_OUT), jnp.float32),
        pltpu.VMEM((ROWS_PER_TILE,), jnp.float32),
        pltpu.VMEM_SHARED((NPAD, D_OUT), jnp.float32),
        pltpu.SemaphoreType.DMA,
    ],
    compiler_params=_SC_PARAMS_NL,
)
def _deg(dst_hbm, zeros_hbm, ones_hbm, out_hbm,
         dst_v, buf_v, rows_v, packed_v, deg_sh, sem):
    cid = lax.axis_index("c")
    sid = lax.axis_index("s")
    wid = sid * NC + cid

    pltpu.sync_copy(zeros_hbm, buf_v)
    for b in range(COPIES_PER_TILE):
        r0 = sid * ROWS_PER_TILE + b * CHUNK
        pltpu.sync_copy(buf_v, deg_sh.at[pl.ds(r0, CHUNK), :])

    pltpu.sync_copy(dst_hbm.at[wid], dst_v)
    pltpu.sync_copy(ones_hbm, buf_v)
    plsc.subcore_barrier()

    def fire(k, carry):
        pltpu.async_copy(buf_v, deg_sh.at[dst_v.at[k]], sem, add=True)
        return carry

    lax.fori_loop(0, K_CHUNKS, fire, 0)

    def drain(k, carry):
        pltpu.make_async_copy(buf_v, deg_sh.at[dst_v.at[0]], sem).wait()
        return carry

    lax.fori_loop(0, K_CHUNKS, drain, 0)
    plsc.subcore_barrier()

    pltpu.sync_copy(deg_sh.at[pl.ds(sid * ROWS_PER_TILE, ROWS_PER_TILE), :],
                    rows_v)

    def compact(j, carry):
        rows = j * 16 + lax.iota(jnp.int32, 16)
        vals = plsc.load_gather(rows_v, [rows, jnp.zeros((16,), jnp.int32)])
        packed_v[pl.ds(j * 16, 16)] = vals
        return carry

    lax.fori_loop(0, _GROUPS_PER_TILE, compact, 0)
    pltpu.sync_copy(packed_v,
                    out_hbm.at[cid, pl.ds(sid * ROWS_PER_TILE, ROWS_PER_TILE)])


BR = 2560
GRID = NPAD // BR
BRH = 2000


def _row_spec(width, rows=BR):
    return pl.BlockSpec((rows, width), lambda i: (i, 0))


def _full_spec(shape):
    return pl.BlockSpec(shape, lambda i: (0,) * len(shape))


def _gmm1_body(degw_ref, x_ref, w_ref, g1_ref):
    dinv = lax.rsqrt(degw_ref[...][:, :1])
    g1_ref[...] = jnp.dot(x_ref[...], w_ref[...],
                          preferred_element_type=jnp.float32) * dinv


_gmm1 = pl.pallas_call(
    _gmm1_body,
    grid=(GRID,),
    in_specs=[_row_spec(D_HID), _row_spec(D_IN), _full_spec((D_IN, D_HID))],
    out_specs=_row_spec(D_HID),
    out_shape=jax.ShapeDtypeStruct((NPAD, D_HID), jnp.float32))


def _layer2_body(acca_ref, accb_ref, g1_ref, degw_ref, b1_ref, w2_ref,
                 g2_ref):
    dinv = lax.rsqrt(degw_ref[...][:, :1])
    z = dinv * (acca_ref[...] + accb_ref[...] + g1_ref[...]) + b1_ref[...]
    z = jnp.maximum(z, 0.0)
    g2_ref[...] = dinv * jnp.dot(z, w2_ref[...],
                                 preferred_element_type=jnp.float32)


_layer2 = pl.pallas_call(
    _layer2_body,
    grid=(GRID,),
    in_specs=[_row_spec(D_HID), _row_spec(D_HID), _row_spec(D_HID),
              _row_spec(D_HID), _full_spec((1, D_HID)),
              _full_spec((D_HID, D_OUT))],
    out_specs=_row_spec(D_OUT),
    out_shape=jax.ShapeDtypeStruct((NPAD, D_OUT), jnp.float32))


def _head_body(acca_ref, accb_ref, g2_ref, degw_ref, b2_ref, soft_ref):
    dinv = lax.rsqrt(degw_ref[...][:, :1])
    logits = dinv * (acca_ref[...] + accb_ref[...] + g2_ref[...]) + b2_ref[...]
    cols = lax.broadcasted_iota(jnp.int32, (BRH, D_OUT), 1)
    alpha = jnp.where(cols < N_CLASSES, 1.0 + jnp.exp(logits), 0.0)
    soft = alpha / jnp.sum(alpha, axis=1, keepdims=True)
    soft_ref[...] = soft[:, :N_CLASSES]


_head = pl.pallas_call(
    _head_body,
    grid=(N_NODES // BRH,),
    in_specs=[_row_spec(D_OUT, BRH), _row_spec(D_OUT, BRH),
              _row_spec(D_OUT, BRH), _row_spec(D_HID, BRH),
              _full_spec((1, D_OUT))],
    out_specs=_row_spec(N_CLASSES, BRH),
    out_shape=jax.ShapeDtypeStruct((N_NODES, N_CLASSES), jnp.float32))


_ZEROS16 = np.zeros((CHUNK, D_OUT), np.float32)
_ONES16 = np.ones((CHUNK, D_OUT), np.float32)
_ZEROS64 = np.zeros((CHUNK, D_HID), np.float32)


def kernel(x, edge_index, W1, b1, W2, b2):
    ei = edge_index.astype(jnp.int32)
    src3 = ei[0].reshape(NW, K_CHUNKS, CHUNK)
    dst3 = ei[1].reshape(NW, K_CHUNKS, CHUNK)

    x_pad = jnp.pad(x, ((0, NPAD - N_NODES), (0, 0)))
    w2p = jnp.pad(W2, ((0, 0), (0, D_OUT - N_CLASSES)))
    b1r = b1.reshape(1, D_HID)
    b2r = jnp.pad(b2, (0, D_OUT - N_CLASSES)).reshape(1, D_OUT)

    degp = _deg(dst3, _ZEROS16, _ONES16)
    degw = jnp.broadcast_to((degp[0] + degp[1] + 1.0)[:, None],
                            (NPAD, D_HID))
    g1 = _gmm1(degw, x_pad, W1)
    acc1 = _agg64(g1, src3, dst3, _ZEROS64)
    g2 = _layer2(acc1[0], acc1[1], g1, degw, b1r, w2p)
    acc2 = _agg16(g2, src3, dst3, _ZEROS16)
    return _head(acc2[0], acc2[1], g2, degw, b2r)


# speedup vs baseline: 1.0560x; 1.0560x over previous
import functools

import jax
import jax.numpy as jnp
import numpy as np
from jax import lax
from jax.experimental import pallas as pl
from jax.experimental.pallas import tpu as pltpu
from jax.experimental.pallas import tpu_sc as plsc

N_NODES = 10000
NPAD = 10240
D_IN = 128
D_HID = 64
D_OUT = 16
N_CLASSES = 10
N_EDGES = 320000

NC, NS = 2, 16
NW = NC * NS
CHUNK = 80
K_CHUNKS = 125
NBUF = 10

ROWS_PER_TILE = NPAD // NS
COPIES_PER_TILE = ROWS_PER_TILE // CHUNK

_MESH = plsc.VectorSubcoreMesh(core_axis_name="c", subcore_axis_name="s")
_SC_PARAMS = pltpu.CompilerParams(use_tc_tiling_on_sc=False)
_SC_PARAMS_NL = pltpu.CompilerParams(use_tc_tiling_on_sc=False,
                                     needs_layout_passes=False)


def _make_agg(width):
    @functools.partial(
        pl.kernel,
        out_type=(jax.ShapeDtypeStruct((NPAD, width), jnp.float32),
                  jax.ShapeDtypeStruct((NPAD, width), jnp.float32)),
        mesh=_MESH,
        scratch_types=[
            pltpu.VMEM((K_CHUNKS, CHUNK), jnp.int32),
            pltpu.VMEM((K_CHUNKS, CHUNK), jnp.int32),
            [pltpu.VMEM((CHUNK, width), jnp.float32) for _ in range(NBUF)],
            pltpu.VMEM_SHARED((NPAD, width), jnp.float32),
            [pltpu.SemaphoreType.DMA for _ in range(NBUF)],
            [pltpu.SemaphoreType.DMA for _ in range(NBUF)],
        ],
        compiler_params=_SC_PARAMS,
    )
    def agg(g_hbm, ei_hbm, zeros_hbm, outa_hbm, outb_hbm,
            src_v, dst_v, bufs, acc_sh, gsems, ssems):
        cid = lax.axis_index("c")
        sid = lax.axis_index("s")
        wid = sid * NC + cid

        pltpu.sync_copy(zeros_hbm, bufs[0])
        for b in range(COPIES_PER_TILE):
            r0 = sid * ROWS_PER_TILE + b * CHUNK
            pltpu.sync_copy(bufs[0], acc_sh.at[pl.ds(r0, CHUNK), :])

        pltpu.sync_copy(ei_hbm.at[0, wid], src_v)
        pltpu.sync_copy(ei_hbm.at[1, wid], dst_v)

        for b in range(NBUF):
            pltpu.async_copy(g_hbm.at[src_v.at[b]], bufs[b], gsems[b])
        plsc.subcore_barrier()

        def rnd(i, carry):
            for b in range(NBUF):
                c = i * NBUF + b
                pltpu.make_async_copy(
                    g_hbm.at[src_v.at[c]], bufs[b], gsems[b]).wait()
                pltpu.async_copy(
                    bufs[b], acc_sh.at[dst_v.at[c]], ssems[b], add=True)
            for b in range(NBUF):
                c2 = (i + 1) * NBUF + b
                pltpu.make_async_copy(
                    bufs[b], acc_sh.at[dst_v.at[0]], ssems[b]).wait()

                @pl.when(c2 < K_CHUNKS)
                def _():
                    pltpu.async_copy(g_hbm.at[src_v.at[c2]], bufs[b], gsems[b])
            return carry

        lax.fori_loop(0, K_CHUNKS // NBUF, rnd, 0)
        for b in range(K_CHUNKS % NBUF):
            c = (K_CHUNKS // NBUF) * NBUF + b
            pltpu.make_async_copy(
                g_hbm.at[src_v.at[c]], bufs[b], gsems[b]).wait()
            pltpu.async_copy(
                bufs[b], acc_sh.at[dst_v.at[c]], ssems[b], add=True)
        for b in range(K_CHUNKS % NBUF):
            pltpu.make_async_copy(
                bufs[b], acc_sh.at[dst_v.at[0]], ssems[b]).wait()
        plsc.subcore_barrier()

        for b in range(COPIES_PER_TILE):
            r0 = sid * ROWS_PER_TILE + b * CHUNK
            pltpu.sync_copy(acc_sh.at[pl.ds(r0, CHUNK), :], bufs[0])

            @pl.when(cid == 0)
            def _():
                pltpu.sync_copy(bufs[0], outa_hbm.at[pl.ds(r0, CHUNK), :])

            @pl.when(cid == 1)
            def _():
                pltpu.sync_copy(bufs[0], outb_hbm.at[pl.ds(r0, CHUNK), :])

    return agg


_agg64 = _make_agg(D_HID)
_agg16 = _make_agg(D_OUT)

_GROUPS_PER_TILE = ROWS_PER_TILE // 16


@functools.partial(
    pl.kernel,
    out_type=jax.ShapeDtypeStruct((NC, NPAD), jnp.float32),
    mesh=_MESH,
    scratch_types=[
        pltpu.VMEM((K_CHUNKS, CHUNK), jnp.int32),
        pltpu.VMEM((CHUNK, D_OUT), jnp.float32),
        pltpu.VMEM((ROWS_PER_TILE, D_OUT), jnp.float32),
        pltpu.VMEM((ROWS_PER_TILE,), jnp.float32),
        pltpu.VMEM_SHARED((NPAD, D_OUT), jnp.float32),
        pltpu.SemaphoreType.DMA,
    ],
    compiler_params=_SC_PARAMS_NL,
)
def _deg(ei_hbm, zeros_hbm, ones_hbm, out_hbm,
         dst_v, buf_v, rows_v, packed_v, deg_sh, sem):
    cid = lax.axis_index("c")
    sid = lax.axis_index("s")
    wid = sid * NC + cid

    pltpu.sync_copy(zeros_hbm, buf_v)
    for b in range(COPIES_PER_TILE):
        r0 = sid * ROWS_PER_TILE + b * CHUNK
        pltpu.sync_copy(buf_v, deg_sh.at[pl.ds(r0, CHUNK), :])

    pltpu.sync_copy(ei_hbm.at[1, wid], dst_v)
    pltpu.sync_copy(ones_hbm, buf_v)
    plsc.subcore_barrier()

    def fire(k, carry):
        pltpu.async_copy(buf_v, deg_sh.at[dst_v.at[k]], sem, add=True)
        return carry

    lax.fori_loop(0, K_CHUNKS, fire, 0)

    def drain(k, carry):
        pltpu.make_async_copy(buf_v, deg_sh.at[dst_v.at[0]], sem).wait()
        return carry

    lax.fori_loop(0, K_CHUNKS, drain, 0)
    plsc.subcore_barrier()

    pltpu.sync_copy(deg_sh.at[pl.ds(sid * ROWS_PER_TILE, ROWS_PER_TILE), :],
                    rows_v)

    def compact(j, carry):
        rows = j * 16 + lax.iota(jnp.int32, 16)
        vals = plsc.load_gather(rows_v, [rows, jnp.zeros((16,), jnp.int32)])
        packed_v[pl.ds(j * 16, 16)] = vals
        return carry

    lax.fori_loop(0, _GROUPS_PER_TILE, compact, 0)
    pltpu.sync_copy(packed_v,
                    out_hbm.at[cid, pl.ds(sid * ROWS_PER_TILE, ROWS_PER_TILE)])


BR = 2560
GRID = NPAD // BR
BRH = 2000


def _row_spec(width, rows=BR):
    return pl.BlockSpec((rows, width), lambda i: (i, 0))


def _full_spec(shape):
    return pl.BlockSpec(shape, lambda i: (0,) * len(shape))


def _gmm1_body(degw_ref, x_ref, w_ref, g1_ref):
    dinv = lax.rsqrt(degw_ref[...][:, :1])
    g1_ref[...] = jnp.dot(x_ref[...], w_ref[...],
                          preferred_element_type=jnp.float32) * dinv


_gmm1 = pl.pallas_call(
    _gmm1_body,
    grid=(GRID,),
    in_specs=[_row_spec(D_HID), _row_spec(D_IN), _full_spec((D_IN, D_HID))],
    out_specs=_row_spec(D_HID),
    out_shape=jax.ShapeDtypeStruct((NPAD, D_HID), jnp.float32))


def _layer2_body(acca_ref, accb_ref, g1_ref, degw_ref, b1_ref, w2_ref,
                 g2_ref):
    dinv = lax.rsqrt(degw_ref[...][:, :1])
    z = dinv * (acca_ref[...] + accb_ref[...] + g1_ref[...]) + b1_ref[...]
    z = jnp.maximum(z, 0.0)
    g2_ref[...] = dinv * jnp.dot(z, w2_ref[...],
                                 preferred_element_type=jnp.float32)


_layer2 = pl.pallas_call(
    _layer2_body,
    grid=(GRID,),
    in_specs=[_row_spec(D_HID), _row_spec(D_HID), _row_spec(D_HID),
              _row_spec(D_HID), _full_spec((1, D_HID)),
              _full_spec((D_HID, D_OUT))],
    out_specs=_row_spec(D_OUT),
    out_shape=jax.ShapeDtypeStruct((NPAD, D_OUT), jnp.float32))


def _head_body(acca_ref, accb_ref, g2_ref, degw_ref, b2_ref, soft_ref):
    dinv = lax.rsqrt(degw_ref[...][:, :1])
    logits = dinv * (acca_ref[...] + accb_ref[...] + g2_ref[...]) + b2_ref[...]
    cols = lax.broadcasted_iota(jnp.int32, (BRH, D_OUT), 1)
    alpha = jnp.where(cols < N_CLASSES, 1.0 + jnp.exp(logits), 0.0)
    soft_ref[...] = alpha / jnp.sum(alpha, axis=1, keepdims=True)


_head = pl.pallas_call(
    _head_body,
    grid=(N_NODES // BRH,),
    in_specs=[_row_spec(D_OUT, BRH), _row_spec(D_OUT, BRH),
              _row_spec(D_OUT, BRH), _row_spec(D_HID, BRH),
              _full_spec((1, D_OUT))],
    out_specs=_row_spec(D_OUT, BRH),
    out_shape=jax.ShapeDtypeStruct((N_NODES, D_OUT), jnp.float32))


_ZEROS16 = np.zeros((CHUNK, D_OUT), np.float32)
_ONES16 = np.ones((CHUNK, D_OUT), np.float32)
_ZEROS64 = np.zeros((CHUNK, D_HID), np.float32)


def kernel(x, edge_index, W1, b1, W2, b2):
    ei4 = edge_index.astype(jnp.int32).reshape(2, NW, K_CHUNKS, CHUNK)

    x_pad = jnp.pad(x, ((0, NPAD - N_NODES), (0, 0)))
    w2p = jnp.pad(W2, ((0, 0), (0, D_OUT - N_CLASSES)))
    b1r = b1.reshape(1, D_HID)
    b2r = jnp.pad(b2, (0, D_OUT - N_CLASSES)).reshape(1, D_OUT)

    degp = _deg(ei4, _ZEROS16, _ONES16)
    degw = jnp.broadcast_to((degp[0] + degp[1] + 1.0)[:, None],
                            (NPAD, D_HID))
    g1 = _gmm1(degw, x_pad, W1)
    acc1 = _agg64(g1, ei4, _ZEROS64)
    g2 = _layer2(acc1[0], acc1[1], g1, degw, b1r, w2p)
    acc2 = _agg16(g2, ei4, _ZEROS16)
    soft = _head(acc2[0], acc2[1], g2, degw, b2r)
    return soft[:, :N_CLASSES]
